# Initial kernel scaffold; baseline (speedup 1.0000x reference)
import jax
import jax.numpy as jnp
from jax.experimental import pallas as pl


def kernel(x, table, scale):
    raise NotImplementedError("write your pallas kernel here")


# speedup vs baseline: 113.3031x; 113.3031x over previous
import functools

import jax
import jax.numpy as jnp
from jax import lax
from jax.experimental import pallas as pl
from jax.experimental.pallas import tpu as pltpu
from jax.experimental.pallas import tpu_sc as plsc

_NUM_EMBEDDINGS = 1000000
_MIN_SCALE = 2.5
_MAX_SCALE = 3.5

_ROWS = 16384
_COLS = 200
_TOTAL = _ROWS * _COLS

_info = plsc.get_sparse_core_info()
_NC = _info.num_cores
_NS = _info.num_subcores
_NW = _NC * _NS
_PER_W = _TOTAL // _NW

_CHUNK = 1024
_SEG = 128
_K = _CHUNK // _SEG
_STEPS = _PER_W // _CHUNK

_mesh = plsc.VectorSubcoreMesh(core_axis_name="c", subcore_axis_name="s")


@functools.partial(
    pl.kernel,
    mesh=_mesh,
    out_type=jax.ShapeDtypeStruct((_TOTAL,), jnp.float32),
    scratch_types=[
        pltpu.VMEM((_CHUNK,), jnp.float32),
        pltpu.VMEM((_CHUNK,), jnp.int32),
        pltpu.VMEM((_CHUNK,), jnp.int32),
        pltpu.VMEM((_CHUNK,), jnp.float32),
        pltpu.VMEM((_CHUNK,), jnp.float32),
        pltpu.VMEM((_CHUNK,), jnp.float32),
        pltpu.VMEM((_CHUNK,), jnp.float32),
        pltpu.SemaphoreType.DMA,
    ],
)
def _sc_remap(o3_hbm, table_hbm, out_hbm,
              o3_v, ilo_v, ihi_v, frac_v, lo_v, hi_v, res_v, sem):
    wid = lax.axis_index("s") * _NC + lax.axis_index("c")
    base = wid * _PER_W

    def step(i, carry):
        off = base + i * _CHUNK
        pltpu.sync_copy(o3_hbm.at[pl.ds(off, _CHUNK)], o3_v)

        def derive(j, c):
            o3 = o3_v[pl.ds(j * 16, 16)]
            li = o3.astype(jnp.int32)
            lf = li.astype(jnp.float32)
            fr = o3 - lf
            ui = li + jnp.where(fr > 0.0, 1, 0)
            ilo_v[pl.ds(j * 16, 16)] = li
            ihi_v[pl.ds(j * 16, 16)] = ui
            frac_v[pl.ds(j * 16, 16)] = fr
            return c

        lax.fori_loop(0, _CHUNK // 16, derive, 0, unroll=True)

        copies = []
        for k in range(_K):
            sl = pl.ds(k * _SEG, _SEG)
            copies.append(pltpu.async_copy(
                table_hbm.at[ilo_v.at[sl]], lo_v.at[sl], sem))
            copies.append(pltpu.async_copy(
                table_hbm.at[ihi_v.at[sl]], hi_v.at[sl], sem))
        for cp in copies:
            cp.wait()

        def interp(j, c):
            sl = pl.ds(j * 16, 16)
            fr = frac_v[sl]
            res_v[sl] = fr * lo_v[sl] + (1.0 - fr) * hi_v[sl]
            return c

        lax.fori_loop(0, _CHUNK // 16, interp, 0, unroll=True)

        pltpu.sync_copy(res_v, out_hbm.at[pl.ds(off, _CHUNK)])
        return carry

    lax.fori_loop(0, _STEPS, step, 0)


def kernel(x, table, scale):
    s = jnp.clip(scale, _MIN_SCALE, _MAX_SCALE)
    mean = jnp.mean(x)
    std = jnp.std(x, ddof=1)
    out = (x - mean) / std
    out_01 = (jnp.clip(out, -s, s) / s + 1.0) / 2.0
    out3 = out_01 * (_NUM_EMBEDDINGS - 1)
    res = _sc_remap(out3.reshape(-1), table.reshape(-1))
    return res.reshape(_ROWS, _COLS)
